# baseline (speedup 1.0000x reference)
import dataclasses
import functools

import jax
import jax.numpy as jnp
from jax import lax
from jax.experimental import pallas as pl
from jax.experimental.pallas import tpu as pltpu
from jax.experimental.pallas import tpu_sc as plsc

N = 10000
E = 320000
NC = 2
NS = 16
NW = NC * NS
EPT_DEG = E // NW
C = 64
EPAD = 327680
EPT = EPAD // NW
NCHUNK = EPT // C
NBUF = 4
NIDX = 2 * NBUF
NSB = NCHUNK // NIDX
NPAD = 10240
ROWS_PT = NPAD // NS

_MESH = plsc.VectorSubcoreMesh(core_axis_name="c", subcore_axis_name="s")
_NO_LAYOUT = dataclasses.replace(pltpu.CompilerParams(),
                                 needs_layout_passes=False)


def _sc_degree(dst):
    @functools.partial(
        pl.kernel,
        out_type=jax.ShapeDtypeStruct((NW * NPAD,), jnp.float32),
        mesh=_MESH,
        scratch_types=[
            pltpu.VMEM((EPT_DEG,), jnp.int32),
            pltpu.VMEM((NPAD,), jnp.float32),
        ],
        compiler_params=_NO_LAYOUT,
    )
    def k(dst_hbm, out_hbm, dbuf, acc):
        cid = lax.axis_index("c")
        sid = lax.axis_index("s")
        wid = sid * NC + cid

        @pl.loop(0, NPAD // 16)
        def _(i):
            acc[pl.ds(i * 16, 16)] = jnp.zeros((16,), jnp.float32)

        pltpu.sync_copy(dst_hbm.at[pl.ds(wid * EPT_DEG, EPT_DEG)], dbuf)

        @pl.loop(0, EPT_DEG // 16)
        def _(i):
            vals = dbuf[pl.ds(i * 16, 16)]
            plsc.addupdate_scatter(acc, [vals], jnp.ones((16,), jnp.float32))

        pltpu.sync_copy(acc, out_hbm.at[pl.ds(wid * NPAD, NPAD)])

    return k(dst)


def _sc_propagate(g, idx2, zrows, d):
    @functools.partial(
        pl.kernel,
        out_type=jax.ShapeDtypeStruct((NC * NPAD, d), jnp.float32),
        mesh=_MESH,
        scratch_types=[
            [pltpu.VMEM((2, C), jnp.int32) for _ in range(NIDX)],
            [pltpu.VMEM((C, d), jnp.float32) for _ in range(NBUF)],
            pltpu.VMEM_SHARED((NPAD, d), jnp.float32),
            [pltpu.SemaphoreType.DMA for _ in range(NIDX)],
            [pltpu.SemaphoreType.DMA for _ in range(NBUF)],
            [pltpu.SemaphoreType.DMA for _ in range(NBUF)],
        ],
    )
    def k(g_hbm, idx_hbm, z_hbm, out_hbm, idx8, rows, acc, si, sg, ss):
        cid = lax.axis_index("c")
        sid = lax.axis_index("s")
        wid = sid * NC + cid
        row0 = sid * ROWS_PT
        base = wid * NCHUNK
        pltpu.sync_copy(z_hbm, acc.at[pl.ds(row0, ROWS_PT)])
        plsc.subcore_barrier()

        for s in range(NIDX):
            pltpu.async_copy(idx_hbm.at[base + s], idx8[s], si[s])
        for t in range(NBUF):
            pltpu.make_async_copy(idx_hbm.at[base + t], idx8[t], si[t]).wait()
            pltpu.async_copy(g_hbm.at[idx8[t].at[0]], rows[t], sg[t])

        @pl.loop(0, NSB)
        def _(sb):
            j0 = sb * NIDX
            for t in range(NBUF):
                j = j0 + t
                pltpu.make_async_copy(g_hbm.at[idx8[t].at[0]], rows[t],
                                      sg[t]).wait()
                pltpu.async_copy(rows[t], acc.at[idx8[t].at[1]], ss[t],
                                 add=True)
            for t in range(NBUF):
                j = j0 + t
                pltpu.make_async_copy(rows[t], acc.at[idx8[t].at[1]],
                                      ss[t]).wait()

                @pl.when(j + NIDX < NCHUNK)
                def _():
                    pltpu.async_copy(idx_hbm.at[base + j + NIDX], idx8[t],
                                     si[t])
                pltpu.make_async_copy(idx_hbm.at[base + j], idx8[t + NBUF],
                                      si[t + NBUF]).wait()
                pltpu.async_copy(g_hbm.at[idx8[t + NBUF].at[0]], rows[t],
                                 sg[t])
            for t in range(NBUF):
                j = j0 + NBUF + t
                pltpu.make_async_copy(g_hbm.at[idx8[t + NBUF].at[0]], rows[t],
                                      sg[t]).wait()
                pltpu.async_copy(rows[t], acc.at[idx8[t + NBUF].at[1]], ss[t],
                                 add=True)
            for t in range(NBUF):
                j = j0 + NBUF + t
                pltpu.make_async_copy(rows[t], acc.at[idx8[t + NBUF].at[1]],
                                      ss[t]).wait()

                @pl.when(j + NIDX < NCHUNK)
                def _():
                    pltpu.async_copy(idx_hbm.at[base + j + NIDX],
                                     idx8[t + NBUF], si[t + NBUF])

                @pl.when(j + NBUF < NCHUNK)
                def _():
                    pltpu.make_async_copy(idx_hbm.at[base + j], idx8[t],
                                          si[t]).wait()
                    pltpu.async_copy(g_hbm.at[idx8[t].at[0]], rows[t], sg[t])

        plsc.subcore_barrier()
        pltpu.sync_copy(
            acc.at[pl.ds(row0, ROWS_PT)],
            out_hbm.at[pl.ds(cid * NPAD + row0, ROWS_PT)],
        )

    return k(g, idx2, zrows)


def _dot(a, b):
    return jnp.dot(a, b, preferred_element_type=jnp.float32,
                   precision=lax.Precision.HIGHEST)


_BN = 1024


def _tc_disg(degp, x, w):
    def body(dp_ref, x_ref, w_ref, dis_ref, g_ref):
        deg = jnp.sum(dp_ref[...], axis=0) + 1.0
        dis = lax.rsqrt(deg)
        dis_ref[...] = dis
        g_ref[...] = _dot(x_ref[...], w_ref[...]) * dis[:, None]

    return pl.pallas_call(
        body,
        grid=(NPAD // _BN,),
        in_specs=[
            pl.BlockSpec((NW, _BN), lambda i: (0, i)),
            pl.BlockSpec((_BN, x.shape[1]), lambda i: (i, 0)),
            pl.BlockSpec((w.shape[0], w.shape[1]), lambda i: (0, 0)),
        ],
        out_specs=(
            pl.BlockSpec((_BN,), lambda i: (i,)),
            pl.BlockSpec((_BN, w.shape[1]), lambda i: (i, 0)),
        ),
        out_shape=(
            jax.ShapeDtypeStruct((NPAD,), jnp.float32),
            jax.ShapeDtypeStruct((NPAD, w.shape[1]), jnp.float32),
        ),
    )(degp, x, w)


def _tc_mid(p, g, dis, b, w):
    d = g.shape[1]

    def body(p0_ref, p1_ref, g_ref, dis_ref, b_ref, w_ref, o_ref):
        s = p0_ref[...] + p1_ref[...] + g_ref[...]
        dis = dis_ref[...]
        h = jnp.maximum(dis[:, None] * s + b_ref[...][None, :], 0.0)
        o_ref[...] = _dot(h, w_ref[...]) * dis[:, None]

    return pl.pallas_call(
        body,
        grid=(NPAD // _BN,),
        in_specs=[
            pl.BlockSpec((_BN, d), lambda i: (i, 0)),
            pl.BlockSpec((_BN, d), lambda i: (i + NPAD // _BN, 0)),
            pl.BlockSpec((_BN, d), lambda i: (i, 0)),
            pl.BlockSpec((_BN,), lambda i: (i,)),
            pl.BlockSpec((d,), lambda i: (0,)),
            pl.BlockSpec((d, w.shape[1]), lambda i: (0, 0)),
        ],
        out_specs=pl.BlockSpec((_BN, w.shape[1]), lambda i: (i, 0)),
        out_shape=jax.ShapeDtypeStruct((NPAD, w.shape[1]), jnp.float32),
    )(p, p, g, dis, b, w)


def _tc_final(p, g, dis, b):
    d_out = b.shape[0]
    d = g.shape[1]

    def body(p0_ref, p1_ref, g_ref, dis_ref, b_ref, o_ref):
        s = (p0_ref[...] + p1_ref[...] + g_ref[...])[:, :d_out]
        dis = dis_ref[...]
        o_ref[...] = jax.nn.sigmoid(dis[:, None] * s + b_ref[...][None, :])

    return pl.pallas_call(
        body,
        grid=(NPAD // _BN,),
        in_specs=[
            pl.BlockSpec((_BN, d), lambda i: (i, 0)),
            pl.BlockSpec((_BN, d), lambda i: (i + NPAD // _BN, 0)),
            pl.BlockSpec((_BN, d), lambda i: (i, 0)),
            pl.BlockSpec((_BN,), lambda i: (i,)),
            pl.BlockSpec((d_out,), lambda i: (0,)),
        ],
        out_specs=pl.BlockSpec((_BN, d_out), lambda i: (i, 0)),
        out_shape=jax.ShapeDtypeStruct((NPAD, d_out), jnp.float32),
    )(p, p, g, dis, b)


def kernel(x, edge_index, W1, b1, W2, b2, W3, b3):
    src = edge_index[0]
    dst = edge_index[1]
    pad = EPAD - E
    pad_src = (jnp.arange(pad, dtype=jnp.int32) * 13) % N
    pad_dst = N + jnp.arange(pad, dtype=jnp.int32) % (NPAD - N)
    srcp = jnp.concatenate([src, pad_src]).reshape(NW, NCHUNK, C)
    dstp = jnp.concatenate([dst, pad_dst]).reshape(NW, NCHUNK, C)
    idx2 = jnp.stack([srcp, dstp], axis=2).reshape(NW * NCHUNK, 2, C)
    z128 = jnp.zeros((ROWS_PT, 128), jnp.float32)
    W3p = jnp.concatenate(
        [W3, jnp.zeros((W3.shape[0], 128 - W3.shape[1]), jnp.float32)], axis=1)

    x_pad = jnp.concatenate(
        [x, jnp.zeros((NPAD - N, x.shape[1]), jnp.float32)], axis=0)

    degp = _sc_degree(dst).reshape(NW, NPAD)
    dis, g1 = _tc_disg(degp, x_pad, W1)

    p1 = _sc_propagate(g1, idx2, z128, 128)
    g2 = _tc_mid(p1, g1, dis, b1, W2)
    p2 = _sc_propagate(g2, idx2, z128, 128)
    g3 = _tc_mid(p2, g2, dis, b2, W3p)
    p3 = _sc_propagate(g3, idx2, z128, 128)
    return _tc_final(p3, g3, dis, b3)[:N]


# speedup vs baseline: 1.0253x; 1.0253x over previous
import dataclasses
import functools

import jax
import jax.numpy as jnp
from jax import lax
from jax.experimental import pallas as pl
from jax.experimental.pallas import tpu as pltpu
from jax.experimental.pallas import tpu_sc as plsc

N = 10000
E = 320000
NC = 2
NS = 16
NW = NC * NS
EPT_DEG = E // NW
C = 64
EPAD = 327680
EPT = EPAD // NW
NCHUNK = EPT // C
NBUF = 4
NIDX = 2 * NBUF
NSB = NCHUNK // NIDX
NPAD = 10240
ROWS_PT = NPAD // NS

_MESH = plsc.VectorSubcoreMesh(core_axis_name="c", subcore_axis_name="s")
_NO_LAYOUT = dataclasses.replace(pltpu.CompilerParams(),
                                 needs_layout_passes=False)


def _sc_degree(dst):
    @functools.partial(
        pl.kernel,
        out_type=jax.ShapeDtypeStruct((NW * N,), jnp.float32),
        mesh=_MESH,
        scratch_types=[
            pltpu.VMEM((EPT_DEG,), jnp.int32),
            pltpu.VMEM((N,), jnp.float32),
        ],
        compiler_params=_NO_LAYOUT,
    )
    def k(dst_hbm, out_hbm, dbuf, acc):
        cid = lax.axis_index("c")
        sid = lax.axis_index("s")
        wid = sid * NC + cid

        @pl.loop(0, N // 16)
        def _(i):
            acc[pl.ds(i * 16, 16)] = jnp.zeros((16,), jnp.float32)

        pltpu.sync_copy(dst_hbm.at[pl.ds(wid * EPT_DEG, EPT_DEG)], dbuf)

        @pl.loop(0, EPT_DEG // 16)
        def _(i):
            vals = dbuf[pl.ds(i * 16, 16)]
            plsc.addupdate_scatter(acc, [vals], jnp.ones((16,), jnp.float32))

        pltpu.sync_copy(acc, out_hbm.at[pl.ds(wid * N, N)])

    return k(dst)


def _sc_propagate(g, idx2, zrows, d):
    @functools.partial(
        pl.kernel,
        out_type=jax.ShapeDtypeStruct((NC * NPAD, d), jnp.float32),
        mesh=_MESH,
        scratch_types=[
            [pltpu.VMEM((2, C), jnp.int32) for _ in range(NIDX)],
            [pltpu.VMEM((C, d), jnp.float32) for _ in range(NBUF)],
            pltpu.VMEM_SHARED((NPAD, d), jnp.float32),
            [pltpu.SemaphoreType.DMA for _ in range(NIDX)],
            [pltpu.SemaphoreType.DMA for _ in range(NBUF)],
            [pltpu.SemaphoreType.DMA for _ in range(NBUF)],
        ],
    )
    def k(g_hbm, idx_hbm, z_hbm, out_hbm, idx8, rows, acc, si, sg, ss):
        cid = lax.axis_index("c")
        sid = lax.axis_index("s")
        wid = sid * NC + cid
        row0 = sid * ROWS_PT
        base = wid * NCHUNK
        pltpu.sync_copy(z_hbm, acc.at[pl.ds(row0, ROWS_PT)])
        plsc.subcore_barrier()

        for s in range(NIDX):
            pltpu.async_copy(idx_hbm.at[base + s], idx8[s], si[s])
        for t in range(NBUF):
            pltpu.make_async_copy(idx_hbm.at[base + t], idx8[t], si[t]).wait()
            pltpu.async_copy(g_hbm.at[idx8[t].at[0]], rows[t], sg[t])

        @pl.loop(0, NSB)
        def _(sb):
            j0 = sb * NIDX
            for t in range(NBUF):
                j = j0 + t
                pltpu.make_async_copy(g_hbm.at[idx8[t].at[0]], rows[t],
                                      sg[t]).wait()
                pltpu.async_copy(rows[t], acc.at[idx8[t].at[1]], ss[t],
                                 add=True)
            for t in range(NBUF):
                j = j0 + t
                pltpu.make_async_copy(rows[t], acc.at[idx8[t].at[1]],
                                      ss[t]).wait()

                @pl.when(j + NIDX < NCHUNK)
                def _():
                    pltpu.async_copy(idx_hbm.at[base + j + NIDX], idx8[t],
                                     si[t])
                pltpu.make_async_copy(idx_hbm.at[base + j], idx8[t + NBUF],
                                      si[t + NBUF]).wait()
                pltpu.async_copy(g_hbm.at[idx8[t + NBUF].at[0]], rows[t],
                                 sg[t])
            for t in range(NBUF):
                j = j0 + NBUF + t
                pltpu.make_async_copy(g_hbm.at[idx8[t + NBUF].at[0]], rows[t],
                                      sg[t]).wait()
                pltpu.async_copy(rows[t], acc.at[idx8[t + NBUF].at[1]], ss[t],
                                 add=True)
            for t in range(NBUF):
                j = j0 + NBUF + t
                pltpu.make_async_copy(rows[t], acc.at[idx8[t + NBUF].at[1]],
                                      ss[t]).wait()

                @pl.when(j + NIDX < NCHUNK)
                def _():
                    pltpu.async_copy(idx_hbm.at[base + j + NIDX],
                                     idx8[t + NBUF], si[t + NBUF])

                @pl.when(j + NBUF < NCHUNK)
                def _():
                    pltpu.make_async_copy(idx_hbm.at[base + j], idx8[t],
                                          si[t]).wait()
                    pltpu.async_copy(g_hbm.at[idx8[t].at[0]], rows[t], sg[t])

        plsc.subcore_barrier()
        pltpu.sync_copy(
            acc.at[pl.ds(row0, ROWS_PT)],
            out_hbm.at[pl.ds(cid * NPAD + row0, ROWS_PT)],
        )

    return k(g, idx2, zrows)


def _dot(a, b):
    return jnp.dot(a, b, preferred_element_type=jnp.float32,
                   precision=lax.Precision.HIGHEST)


def _tc_disg(degp, x, w):
    def body(dp_ref, x_ref, w_ref, dis_ref, g_ref):
        deg = jnp.sum(dp_ref[...], axis=0) + 1.0
        dis = lax.rsqrt(deg)
        dis_ref[...] = dis
        g_ref[...] = _dot(x_ref[...], w_ref[...]) * dis[:, None]

    return pl.pallas_call(
        body,
        out_shape=(
            jax.ShapeDtypeStruct((N,), jnp.float32),
            jax.ShapeDtypeStruct((N, w.shape[1]), jnp.float32),
        ),
    )(degp, x, w)


def _tc_mid(p, g, dis, b, w):
    def body(p_ref, g_ref, dis_ref, b_ref, w_ref, o_ref):
        pr = p_ref[...]
        s = pr[:N] + pr[NPAD:NPAD + N] + g_ref[...]
        dis = dis_ref[...]
        h = jnp.maximum(dis[:, None] * s + b_ref[...][None, :], 0.0)
        o_ref[...] = _dot(h, w_ref[...]) * dis[:, None]

    return pl.pallas_call(
        body,
        out_shape=jax.ShapeDtypeStruct((N, w.shape[1]), jnp.float32),
    )(p, g, dis, b, w)


def _tc_final(p, g, dis, b):
    d_out = b.shape[0]

    def body(p_ref, g_ref, dis_ref, b_ref, o_ref):
        pr = p_ref[...]
        s = (pr[:N] + pr[NPAD:NPAD + N] + g_ref[...])[:, :d_out]
        dis = dis_ref[...]
        o_ref[...] = jax.nn.sigmoid(dis[:, None] * s + b_ref[...][None, :])

    return pl.pallas_call(
        body,
        out_shape=jax.ShapeDtypeStruct((N, d_out), jnp.float32),
    )(p, g, dis, b)


def kernel(x, edge_index, W1, b1, W2, b2, W3, b3):
    src = edge_index[0]
    dst = edge_index[1]
    pad = EPAD - E
    pad_src = (jnp.arange(pad, dtype=jnp.int32) * 13) % N
    pad_dst = N + jnp.arange(pad, dtype=jnp.int32) % (NPAD - N)
    srcp = jnp.concatenate([src, pad_src]).reshape(NW, NCHUNK, C)
    dstp = jnp.concatenate([dst, pad_dst]).reshape(NW, NCHUNK, C)
    idx2 = jnp.stack([srcp, dstp], axis=2).reshape(NW * NCHUNK, 2, C)
    z128 = jnp.zeros((ROWS_PT, 128), jnp.float32)
    W3p = jnp.concatenate(
        [W3, jnp.zeros((W3.shape[0], 128 - W3.shape[1]), jnp.float32)], axis=1)

    degp = _sc_degree(dst).reshape(NW, N)
    dis, g1 = _tc_disg(degp, x, W1)

    p1 = _sc_propagate(g1, idx2, z128, 128)
    g2 = _tc_mid(p1, g1, dis, b1, W2)
    p2 = _sc_propagate(g2, idx2, z128, 128)
    g3 = _tc_mid(p2, g2, dis, b2, W3p)
    p3 = _sc_propagate(g3, idx2, z128, 128)
    return _tc_final(p3, g3, dis, b3)


# speedup vs baseline: 1.0332x; 1.0077x over previous
import dataclasses
import functools

import jax
import jax.numpy as jnp
from jax import lax
from jax.experimental import pallas as pl
from jax.experimental.pallas import tpu as pltpu
from jax.experimental.pallas import tpu_sc as plsc

N = 10000
E = 320000
NC = 2
NS = 16
NW = NC * NS
EPT_DEG = E // NW
C = 64
EPAD = 327680
EPT = EPAD // NW
NCHUNK = EPT // C
NBUF = 4
NIDX = 2 * NBUF
NSB = NCHUNK // NIDX
NPAD = 10240
ROWS_PT = NPAD // NS

_MESH = plsc.VectorSubcoreMesh(core_axis_name="c", subcore_axis_name="s")
_NO_LAYOUT = dataclasses.replace(pltpu.CompilerParams(),
                                 needs_layout_passes=False)


def _sc_degree(dst):
    @functools.partial(
        pl.kernel,
        out_type=jax.ShapeDtypeStruct((NW * N,), jnp.float32),
        mesh=_MESH,
        scratch_types=[
            pltpu.VMEM((EPT_DEG,), jnp.int32),
            pltpu.VMEM((N,), jnp.float32),
        ],
        compiler_params=_NO_LAYOUT,
    )
    def k(dst_hbm, out_hbm, dbuf, acc):
        cid = lax.axis_index("c")
        sid = lax.axis_index("s")
        wid = sid * NC + cid

        @pl.loop(0, N // 16)
        def _(i):
            acc[pl.ds(i * 16, 16)] = jnp.zeros((16,), jnp.float32)

        pltpu.sync_copy(dst_hbm.at[pl.ds(wid * EPT_DEG, EPT_DEG)], dbuf)

        @pl.loop(0, EPT_DEG // 16)
        def _(i):
            vals = dbuf[pl.ds(i * 16, 16)]
            plsc.addupdate_scatter(acc, [vals], jnp.ones((16,), jnp.float32))

        pltpu.sync_copy(acc, out_hbm.at[pl.ds(wid * N, N)])

    return k(dst)


def _sc_propagate(g, idx2, zrows, d):
    @functools.partial(
        pl.kernel,
        out_type=jax.ShapeDtypeStruct((NC * NPAD, d), jnp.float32),
        mesh=_MESH,
        scratch_types=[
            [pltpu.VMEM((2, C), jnp.int32) for _ in range(NIDX)],
            [pltpu.VMEM((C, d), jnp.float32) for _ in range(NBUF)],
            pltpu.VMEM_SHARED((NPAD, d), jnp.float32),
            [pltpu.SemaphoreType.DMA for _ in range(NIDX)],
            [pltpu.SemaphoreType.DMA for _ in range(NBUF)],
            [pltpu.SemaphoreType.DMA for _ in range(NBUF)],
        ],
    )
    def k(g_hbm, idx_hbm, z_hbm, out_hbm, idx8, rows, acc, si, sg, ss):
        cid = lax.axis_index("c")
        sid = lax.axis_index("s")
        wid = sid * NC + cid
        row0 = sid * ROWS_PT
        base = wid * NCHUNK
        for s in range(NIDX):
            pltpu.async_copy(idx_hbm.at[base + s], idx8[s], si[s])
        for t in range(NBUF):
            pltpu.make_async_copy(idx_hbm.at[base + t], idx8[t], si[t]).wait()
            pltpu.async_copy(g_hbm.at[idx8[t].at[0]], rows[t], sg[t])
        pltpu.sync_copy(z_hbm, acc.at[pl.ds(row0, ROWS_PT)])
        plsc.subcore_barrier()

        @pl.loop(0, NSB)
        def _(sb):
            j0 = sb * NIDX
            for t in range(NBUF):
                j = j0 + t
                pltpu.make_async_copy(g_hbm.at[idx8[t].at[0]], rows[t],
                                      sg[t]).wait()
                pltpu.async_copy(rows[t], acc.at[idx8[t].at[1]], ss[t],
                                 add=True)
            for t in range(NBUF):
                j = j0 + t
                pltpu.make_async_copy(rows[t], acc.at[idx8[t].at[1]],
                                      ss[t]).wait()

                @pl.when(j + NIDX < NCHUNK)
                def _():
                    pltpu.async_copy(idx_hbm.at[base + j + NIDX], idx8[t],
                                     si[t])
                pltpu.make_async_copy(idx_hbm.at[base + j], idx8[t + NBUF],
                                      si[t + NBUF]).wait()
                pltpu.async_copy(g_hbm.at[idx8[t + NBUF].at[0]], rows[t],
                                 sg[t])
            for t in range(NBUF):
                j = j0 + NBUF + t
                pltpu.make_async_copy(g_hbm.at[idx8[t + NBUF].at[0]], rows[t],
                                      sg[t]).wait()
                pltpu.async_copy(rows[t], acc.at[idx8[t + NBUF].at[1]], ss[t],
                                 add=True)
            for t in range(NBUF):
                j = j0 + NBUF + t
                pltpu.make_async_copy(rows[t], acc.at[idx8[t + NBUF].at[1]],
                                      ss[t]).wait()

                @pl.when(j + NIDX < NCHUNK)
                def _():
                    pltpu.async_copy(idx_hbm.at[base + j + NIDX],
                                     idx8[t + NBUF], si[t + NBUF])

                @pl.when(j + NBUF < NCHUNK)
                def _():
                    pltpu.make_async_copy(idx_hbm.at[base + j], idx8[t],
                                          si[t]).wait()
                    pltpu.async_copy(g_hbm.at[idx8[t].at[0]], rows[t], sg[t])

        plsc.subcore_barrier()
        pltpu.sync_copy(
            acc.at[pl.ds(row0, ROWS_PT)],
            out_hbm.at[pl.ds(cid * NPAD + row0, ROWS_PT)],
        )

    return k(g, idx2, zrows)


def _dot(a, b):
    return jnp.dot(a, b, preferred_element_type=jnp.float32,
                   precision=lax.Precision.HIGHEST)


def _tc_disg(degp, x, w):
    def body(dp_ref, x_ref, w_ref, dis_ref, g_ref):
        deg = jnp.sum(dp_ref[...], axis=0) + 1.0
        dis = lax.rsqrt(deg)
        dis_ref[...] = dis
        g_ref[...] = _dot(x_ref[...], w_ref[...]) * dis[:, None]

    return pl.pallas_call(
        body,
        out_shape=(
            jax.ShapeDtypeStruct((N,), jnp.float32),
            jax.ShapeDtypeStruct((N, w.shape[1]), jnp.float32),
        ),
    )(degp, x, w)


def _tc_mid(p, g, dis, b, w):
    def body(p_ref, g_ref, dis_ref, b_ref, w_ref, o_ref):
        pr = p_ref[...]
        s = pr[:N] + pr[NPAD:NPAD + N] + g_ref[...]
        dis = dis_ref[...]
        h = jnp.maximum(dis[:, None] * s + b_ref[...][None, :], 0.0)
        o_ref[...] = _dot(h, w_ref[...]) * dis[:, None]

    return pl.pallas_call(
        body,
        out_shape=jax.ShapeDtypeStruct((N, w.shape[1]), jnp.float32),
    )(p, g, dis, b, w)


def _tc_final(p, g, dis, b):
    d_out = b.shape[0]

    def body(p_ref, g_ref, dis_ref, b_ref, o_ref):
        pr = p_ref[...]
        s = (pr[:N] + pr[NPAD:NPAD + N] + g_ref[...])[:, :d_out]
        dis = dis_ref[...]
        o_ref[...] = jax.nn.sigmoid(dis[:, None] * s + b_ref[...][None, :])

    return pl.pallas_call(
        body,
        out_shape=jax.ShapeDtypeStruct((N, d_out), jnp.float32),
    )(p, g, dis, b)


def kernel(x, edge_index, W1, b1, W2, b2, W3, b3):
    src = edge_index[0]
    dst = edge_index[1]
    pad = EPAD - E
    pad_src = (jnp.arange(pad, dtype=jnp.int32) * 13) % N
    pad_dst = N + jnp.arange(pad, dtype=jnp.int32) % (NPAD - N)
    srcp = jnp.concatenate([src, pad_src]).reshape(NW, NCHUNK, C)
    dstp = jnp.concatenate([dst, pad_dst]).reshape(NW, NCHUNK, C)
    idx2 = jnp.stack([srcp, dstp], axis=2).reshape(NW * NCHUNK, 2, C)
    z128 = jnp.zeros((ROWS_PT, 128), jnp.float32)
    W3p = jnp.concatenate(
        [W3, jnp.zeros((W3.shape[0], 128 - W3.shape[1]), jnp.float32)], axis=1)

    degp = _sc_degree(dst).reshape(NW, N)
    dis, g1 = _tc_disg(degp, x, W1)

    p1 = _sc_propagate(g1, idx2, z128, 128)
    g2 = _tc_mid(p1, g1, dis, b1, W2)
    p2 = _sc_propagate(g2, idx2, z128, 128)
    g3 = _tc_mid(p2, g2, dis, b2, W3p)
    p3 = _sc_propagate(g3, idx2, z128, 128)
    return _tc_final(p3, g3, dis, b3)
